# baseline (speedup 1.0000x reference)
import jax
import jax.numpy as jnp
from jax import lax
from jax.experimental import pallas as pl
from jax.experimental.pallas import tpu as pltpu
from jax.experimental.pallas import tpu_sc as plsc

B = 8
N = 16384
C = 3
NPTS = 1024
BIG = 1e10
L = 16
NSH = 4
SHARD = N // NSH
NSL = SHARD // L


def _sc_body(x_hbm, out_hbm,
             xv0, xv1, xv2, dv, ov0, ov1, ov2,
             accv, acci, qv, rdv, rdi, shv, shi):
    c = lax.axis_index("c")
    s = lax.axis_index("s")
    bloc = s // NSH
    shard = s % NSH
    bglob = c * (B // 2) + bloc

    pltpu.sync_copy(x_hbm.at[pl.ds((0 * B + bglob) * N, N)], xv0)
    pltpu.sync_copy(x_hbm.at[pl.ds((1 * B + bglob) * N, N)], xv1)
    pltpu.sync_copy(x_hbm.at[pl.ds((2 * B + bglob) * N, N)], xv2)

    iota = lax.broadcasted_iota(jnp.int32, (L,), 0)
    lane0 = iota == 0
    big = jnp.full((L,), BIG, dtype=jnp.float32)

    def init_d(j, carry):
        dv[pl.ds(j * L, L)] = big
        return carry
    lax.fori_loop(0, NSL, init_d, 0)

    av0 = dv[pl.ds(0, L)]
    m0 = lax.reduce_max(av0, (0,))
    n0 = lax.reduce_min(jnp.where(av0 == m0, iota, N), (0,))
    zero_idx = jnp.full((L,), n0, dtype=jnp.int32)
    for ch, (r, o) in enumerate(((xv0, ov0), (xv1, ov1), (xv2, ov2))):
        q = plsc.load_gather(r, [zero_idx])
        qv[pl.ds(ch * L, L)] = q
        plsc.store_scatter(o, [zero_idx], q, mask=lane0)

    base = shard * SHARD

    def step(i, carry):
        qx = qv[pl.ds(0, L)]
        qy = qv[pl.ds(L, L)]
        qz = qv[pl.ds(2 * L, L)]
        accv[...] = jnp.full((L,), -1.0, jnp.float32)
        acci[...] = jnp.zeros((L,), jnp.int32)

        def scan(j, c2):
            off = j * L
            xs = xv0[pl.ds(base + off, L)]
            ys = xv1[pl.ds(base + off, L)]
            zs = xv2[pl.ds(base + off, L)]
            dx = xs - qx
            dy = ys - qy
            dz = zs - qz
            d = dx * dx + dy * dy + dz * dz
            dn = jnp.minimum(dv[pl.ds(off, L)], d)
            dv[pl.ds(off, L)] = dn
            gi = iota + (base + off)
            av = accv[...]
            ai = acci[...]
            t = dn > av
            accv[...] = jnp.where(t, dn, av)
            acci[...] = jnp.where(t, gi, ai)
            return c2

        lax.fori_loop(0, NSL, scan, 0)
        par = i % 2
        slot = par * (16 * L) + s * L
        pltpu.sync_copy(accv, shv.at[pl.ds(slot, L)])
        pltpu.sync_copy(acci, shi.at[pl.ds(slot, L)])
        plsc.subcore_barrier()
        rdbase = par * (16 * L) + bloc * (NSH * L)
        pltpu.sync_copy(shv.at[pl.ds(rdbase, NSH * L)], rdv)
        pltpu.sync_copy(shi.at[pl.ds(rdbase, NSH * L)], rdi)
        mv = rdv[pl.ds(0, L)]
        mi = rdi[pl.ds(0, L)]
        for k in range(1, NSH):
            bv = rdv[pl.ds(k * L, L)]
            bi = rdi[pl.ds(k * L, L)]
            tb = (bv > mv) | ((bv == mv) & (bi < mi))
            mv = jnp.where(tb, bv, mv)
            mi = jnp.where(tb, bi, mi)
        m = lax.reduce_max(mv, (0,))
        nxt = lax.reduce_min(jnp.where(mv == m, mi, N), (0,))
        idxv = jnp.full((L,), nxt, dtype=jnp.int32)
        posv = jnp.full((L,), i + 1, dtype=jnp.int32)
        for ch, (r, o) in enumerate(((xv0, ov0), (xv1, ov1), (xv2, ov2))):
            q = plsc.load_gather(r, [idxv])
            qv[pl.ds(ch * L, L)] = q
            plsc.store_scatter(o, [posv], q, mask=lane0)
        return carry

    lax.fori_loop(0, NPTS - 1, step, 0)

    @pl.when(shard == 0)
    def _():
        pltpu.sync_copy(ov0, out_hbm.at[pl.ds((bglob * C + 0) * NPTS, NPTS)])
        pltpu.sync_copy(ov1, out_hbm.at[pl.ds((bglob * C + 1) * NPTS, NPTS)])
        pltpu.sync_copy(ov2, out_hbm.at[pl.ds((bglob * C + 2) * NPTS, NPTS)])


def kernel(x):
    xt = jnp.transpose(x, (1, 0, 2)).reshape(C * B * N)
    mesh = plsc.VectorSubcoreMesh(
        core_axis_name="c", subcore_axis_name="s", num_cores=2,
        num_subcores=16,
    )
    f = pl.kernel(
        _sc_body,
        out_type=jax.ShapeDtypeStruct((B * C * NPTS,), jnp.float32),
        mesh=mesh,
        compiler_params=pltpu.CompilerParams(needs_layout_passes=False),
        scratch_types=[
            pltpu.VMEM((N,), jnp.float32),
            pltpu.VMEM((N,), jnp.float32),
            pltpu.VMEM((N,), jnp.float32),
            pltpu.VMEM((SHARD,), jnp.float32),
            pltpu.VMEM((NPTS,), jnp.float32),
            pltpu.VMEM((NPTS,), jnp.float32),
            pltpu.VMEM((NPTS,), jnp.float32),
            pltpu.VMEM((L,), jnp.float32),
            pltpu.VMEM((L,), jnp.int32),
            pltpu.VMEM((3 * L,), jnp.float32),
            pltpu.VMEM((NSH * L,), jnp.float32),
            pltpu.VMEM((NSH * L,), jnp.int32),
            pltpu.VMEM_SHARED((2 * 16 * L,), jnp.float32),
            pltpu.VMEM_SHARED((2 * 16 * L,), jnp.int32),
        ],
    )
    return f(xt).reshape(B, C, NPTS)


# speedup vs baseline: 7.3410x; 7.3410x over previous
import jax
import jax.numpy as jnp
from jax.experimental import pallas as pl
from jax.experimental.pallas import tpu as pltpu

B = 8
N = 16384
C = 3
NPTS = 1024
BIG = 1e10
CH = 512
NCH = N // CH
NACC = 2


def _merge(a, b):
    av, ai, ax, ay, az = a
    bv, bi, bx, by, bz = b
    take_b = (bv > av) | ((bv == av) & (bi < ai))
    return (
        jnp.where(take_b, bv, av),
        jnp.where(take_b, bi, ai),
        jnp.where(take_b, bx, ax),
        jnp.where(take_b, by, ay),
        jnp.where(take_b, bz, az),
    )


def _fps_body(x_ref, out_ref, dists_ref):
    iota_p = jax.lax.broadcasted_iota(jnp.int32, (B, NPTS), 1)
    iota_c = jax.lax.broadcasted_iota(jnp.int32, (B, CH), 1)

    qx0 = x_ref[0, :, 0:1]
    qy0 = x_ref[1, :, 0:1]
    qz0 = x_ref[2, :, 0:1]
    zeros_p = jnp.zeros((B, NPTS), dtype=jnp.float32)
    out_ref[0] = jnp.where(iota_p == 0, qx0, zeros_p)
    out_ref[1] = jnp.where(iota_p == 0, qy0, zeros_p)
    out_ref[2] = jnp.where(iota_p == 0, qz0, zeros_p)
    dists_ref[...] = jnp.full((B, N), BIG, dtype=jnp.float32)

    def body(i, q):
        qx, qy, qz = q
        accs = [None] * NACC
        for c in range(NCH):
            sl = slice(c * CH, (c + 1) * CH)
            xc = x_ref[0, :, sl]
            yc = x_ref[1, :, sl]
            zc = x_ref[2, :, sl]
            dx = xc - qx
            dy = yc - qy
            dz = zc - qz
            d = dx * dx + (dy * dy + dz * dz)
            dn = jnp.minimum(dists_ref[:, sl], d)
            dists_ref[:, sl] = dn
            gi = iota_c + (c * CH)
            k = c % NACC
            if accs[k] is None:
                accs[k] = (dn, gi, xc, yc, zc)
            else:
                av, ai, ax, ay, az = accs[k]
                t = dn > av
                accs[k] = (
                    jnp.where(t, dn, av),
                    jnp.where(t, gi, ai),
                    jnp.where(t, xc, ax),
                    jnp.where(t, yc, ay),
                    jnp.where(t, zc, az),
                )
        acc = accs[0]
        for k in range(1, NACC):
            acc = _merge(acc, accs[k])
        w = CH
        while w > 128:
            h = w // 2
            acc = _merge(tuple(t[:, :h] for t in acc),
                         tuple(t[:, h:] for t in acc))
            w = h
        av, ai, ax, ay, az = acc
        def _lex(a, b):
            (a_v, a_i), (b_v, b_i) = a, b
            tb = (b_v > a_v) | ((b_v == a_v) & (b_i < a_i))
            return jnp.where(tb, b_v, a_v), jnp.where(tb, b_i, a_i)

        def _stage(pair, shifts):
            cands = [pair] + [
                (pltpu.roll(pair[0], s, 1), pltpu.roll(pair[1], s, 1))
                for s in shifts
            ]
            while len(cands) > 1:
                cands = [_lex(cands[j], cands[j + 1])
                         for j in range(0, len(cands) - 1, 2)] + (
                             [cands[-1]] if len(cands) % 2 else [])
            return cands[0]

        p1 = _stage((av, ai), [8 * k for k in range(1, 16)])
        _, nxt = _stage(p1, list(range(1, 8)))
        pos = jnp.bitwise_and(nxt, 127)
        qx = jnp.take_along_axis(ax, pos, axis=1)
        qy = jnp.take_along_axis(ay, pos, axis=1)
        qz = jnp.take_along_axis(az, pos, axis=1)
        osel = iota_p == (i + 1)
        qxp = jnp.concatenate([qx] * (NPTS // 128), axis=1)
        qyp = jnp.concatenate([qy] * (NPTS // 128), axis=1)
        qzp = jnp.concatenate([qz] * (NPTS // 128), axis=1)
        out_ref[0] = jnp.where(osel, qxp, out_ref[0])
        out_ref[1] = jnp.where(osel, qyp, out_ref[1])
        out_ref[2] = jnp.where(osel, qzp, out_ref[2])
        qxc = jnp.concatenate([qx] * (CH // 128), axis=1)
        qyc = jnp.concatenate([qy] * (CH // 128), axis=1)
        qzc = jnp.concatenate([qz] * (CH // 128), axis=1)
        return qxc, qyc, qzc

    q0 = (jnp.broadcast_to(qx0, (B, CH)),
          jnp.broadcast_to(qy0, (B, CH)),
          jnp.broadcast_to(qz0, (B, CH)))
    jax.lax.fori_loop(0, NPTS - 1, body, q0)


def kernel(x):
    xt = jnp.transpose(x, (1, 0, 2))
    out = pl.pallas_call(
        _fps_body,
        out_shape=jax.ShapeDtypeStruct((C, B, NPTS), jnp.float32),
        scratch_shapes=[pltpu.VMEM((B, N), jnp.float32)],
    )(xt)
    return jnp.transpose(out, (1, 0, 2))
